# baseline (speedup 1.0000x reference)
import functools

import jax
import jax.numpy as jnp
from jax import lax
from jax.experimental import pallas as pl
from jax.experimental.pallas import tpu as pltpu
from jax.experimental.pallas import tpu_sc as plsc

L = 16
NW = 32
D = 128
CH = 128


def _sc_embed(emb2d, cat_table, subcat_table, *, n_rows, feat):
    out_w = feat - 2 + 2 * D
    rows_per_w = n_rows // NW
    steps = rows_per_w // CH
    mesh = plsc.VectorSubcoreMesh(core_axis_name="c", subcore_axis_name="s")

    @functools.partial(
        pl.kernel,
        out_type=jax.ShapeDtypeStruct((n_rows, out_w), jnp.float32),
        mesh=mesh,
        scratch_types=[
            pltpu.VMEM((CH, feat), jnp.float32),
            pltpu.VMEM((CH,), jnp.int32),
            pltpu.VMEM((CH,), jnp.int32),
            pltpu.VMEM((CH, D), jnp.float32),
            pltpu.VMEM((CH, D), jnp.float32),
            pltpu.SemaphoreType.DMA,
            pltpu.SemaphoreType.DMA,
        ],
        compiler_params=pltpu.CompilerParams(
            use_tc_tiling_on_sc=False, needs_layout_passes=False),
    )
    def body(emb_hbm, cat_hbm, sub_hbm, out_hbm,
             in_v, ic_v, is_v, cat_v, sub_v, sem_c, sem_s):
        wid = lax.axis_index("s") * 2 + lax.axis_index("c")
        base_w = wid * rows_per_w

        @pl.loop(0, steps)
        def _(t):
            base = base_w + t * CH
            pltpu.sync_copy(emb_hbm.at[pl.ds(base, CH)], in_v)

            @pl.loop(0, CH // L)
            def _(j):
                rows = lax.iota(jnp.int32, L) + j * L
                vc = plsc.load_gather(in_v, [rows, jnp.full((L,), feat - 2, jnp.int32)])
                vs = plsc.load_gather(in_v, [rows, jnp.full((L,), feat - 1, jnp.int32)])
                ic_v[pl.ds(j * L, L)] = vc.astype(jnp.int32)
                is_v[pl.ds(j * L, L)] = vs.astype(jnp.int32)

            dc = pltpu.async_copy(cat_hbm.at[ic_v], cat_v, sem_c)
            ds_ = pltpu.async_copy(sub_hbm.at[is_v], sub_v, sem_s)
            dc.wait()
            ds_.wait()

            pltpu.sync_copy(in_v.at[:, pl.ds(0, feat - 2)],
                            out_hbm.at[pl.ds(base, CH), pl.ds(0, feat - 2)])
            pltpu.sync_copy(cat_v, out_hbm.at[pl.ds(base, CH), pl.ds(feat - 2, D)])
            pltpu.sync_copy(sub_v, out_hbm.at[pl.ds(base, CH), pl.ds(feat - 2 + D, D)])

    return body(emb2d, cat_table, subcat_table)


def kernel(embeddings, cat_table, subcat_table):
    b, s, feat = embeddings.shape
    n_rows = b * s
    emb2d = embeddings.reshape(n_rows, feat)
    out = _sc_embed(emb2d, cat_table, subcat_table, n_rows=n_rows, feat=feat)
    return out.reshape(b, s, feat - 2 + 2 * D)
